# Initial kernel scaffold; baseline (speedup 1.0000x reference)
import functools

import jax
import jax.numpy as jnp
from jax import lax
from jax.experimental import pallas as pl
from jax.experimental.pallas import tpu as pltpu
from jax.experimental.pallas import tpu_sc as plsc

NNODE = 10000
NWORKER = 32
CHUNK = 128
ROWS_PER_SUB = NNODE // 16


def _mm1_body(x_ref, w_ref, o_ref):
    o_ref[...] = jnp.dot(x_ref[...], w_ref[...],
                         preferred_element_type=jnp.float32)


def _mm1(x, W1):
    n, _ = x.shape
    d = W1.shape[1]
    return pl.pallas_call(
        _mm1_body,
        out_shape=jax.ShapeDtypeStruct((n, d), jnp.float32),
    )(x, W1)


def _mm2_body(p_ref, b1_ref, w2_ref, o_ref):
    h = jnp.maximum(p_ref[0] + p_ref[1] + b1_ref[...], 0.0)
    o_ref[...] = jnp.dot(h, w2_ref[...], preferred_element_type=jnp.float32)


def _mm2(p, b1, W2):
    n = p.shape[1]
    d = W2.shape[1]
    return pl.pallas_call(
        _mm2_body,
        out_shape=jax.ShapeDtypeStruct((n, d), jnp.float32),
    )(p, b1.reshape(1, -1), W2)


def _fin_body(q_ref, b2_ref, o_ref):
    o_ref[...] = q_ref[0] + q_ref[1] + b2_ref[...]


def _fin(q, b2):
    n, d = q.shape[1], q.shape[2]
    return pl.pallas_call(
        _fin_body,
        out_shape=jax.ShapeDtypeStruct((n, d), jnp.float32),
    )(q, b2.reshape(1, -1))


def _make_spmm(d, n_chunks):
    nv = d // 16
    mesh = plsc.VectorSubcoreMesh(core_axis_name="c", subcore_axis_name="s")

    @functools.partial(
        pl.kernel,
        out_type=jax.ShapeDtypeStruct((2 * NNODE, d), jnp.float32),
        mesh=mesh,
        scratch_types=[
            pltpu.VMEM((n_chunks, CHUNK), jnp.int32),
            pltpu.VMEM((n_chunks, CHUNK), jnp.int32),
            pltpu.VMEM((n_chunks, CHUNK), jnp.float32),
            pltpu.VMEM((CHUNK, d), jnp.float32),
            pltpu.VMEM_SHARED((NNODE, d), jnp.float32),
            pltpu.SemaphoreType.DMA,
        ],
    )
    def spmm(h_hbm, src_hbm, dst_hbm, w_hbm, out_hbm,
             src_v, dst_v, w_v, gbuf, acc, sem):
        c = lax.axis_index("c")
        s = lax.axis_index("s")
        wid = s * 2 + c

        pltpu.sync_copy(src_hbm.at[wid], src_v)
        pltpu.sync_copy(dst_hbm.at[wid], dst_v)
        pltpu.sync_copy(w_hbm.at[wid], w_v)

        zero16 = jnp.zeros((16,), jnp.float32)

        def zb(i, carry):
            gbuf[i // nv, pl.ds((i % nv) * 16, 16)] = zero16
            return carry

        lax.fori_loop(0, CHUNK * nv, zb, 0)

        base_row = s * ROWS_PER_SUB
        off = 0
        while off < ROWS_PER_SUB:
            rows = min(CHUNK, ROWS_PER_SUB - off)
            pltpu.sync_copy(gbuf.at[pl.ds(0, rows)],
                            acc.at[pl.ds(base_row + off, rows)])
            off += rows
        plsc.subcore_barrier()

        def chunk_body(ci, carry):
            pltpu.async_copy(h_hbm.at[src_v.at[ci]], gbuf, sem).wait()

            def edge_body(e, carry2):
                wv = plsc.load_gather(
                    w_v, [jnp.full((16,), ci, jnp.int32),
                          jnp.full((16,), e, jnp.int32)])
                for j in range(nv):
                    sl = pl.ds(j * 16, 16)
                    gbuf[e, sl] = gbuf[e, sl] * wv
                return carry2

            lax.fori_loop(0, CHUNK, edge_body, 0)
            pltpu.sync_copy(gbuf, acc.at[dst_v.at[ci]], add=True)
            return carry

        lax.fori_loop(0, n_chunks, chunk_body, 0)
        plsc.subcore_barrier()

        off = 0
        while off < ROWS_PER_SUB:
            rows = min(CHUNK, ROWS_PER_SUB - off)
            pltpu.sync_copy(
                acc.at[pl.ds(base_row + off, rows)],
                out_hbm.at[pl.ds(c * NNODE + base_row + off, rows)])
            off += rows

    return spmm


def kernel(x, edge_index, edge_weight, W1, b1, W2, b2):
    src = edge_index[0].astype(jnp.int32)
    dst = edge_index[1].astype(jnp.int32)
    w = edge_weight.astype(jnp.float32)
    e = src.shape[0]

    n_chunks = -(-e // (NWORKER * CHUNK))
    e_pad = NWORKER * n_chunks * CHUNK
    pad = e_pad - e
    if pad:
        src = jnp.concatenate([src, jnp.zeros((pad,), jnp.int32)])
        dst = jnp.concatenate([dst, jnp.zeros((pad,), jnp.int32)])
        w = jnp.concatenate([w, jnp.zeros((pad,), jnp.float32)])
    src3 = src.reshape(NWORKER, n_chunks, CHUNK)
    dst3 = dst.reshape(NWORKER, n_chunks, CHUNK)
    w3 = w.reshape(NWORKER, n_chunks, CHUNK)

    nhid = W1.shape[1]
    ncls = W2.shape[1]

    h = _mm1(x, W1)
    p = _make_spmm(nhid, n_chunks)(h, src3, dst3, w3)
    h2 = _mm2(p.reshape(2, NNODE, nhid), b1, W2)
    q = _make_spmm(ncls, n_chunks)(h2, src3, dst3, w3)
    return _fin(q.reshape(2, NNODE, ncls), b2)


# speedup vs baseline: 6.1530x; 6.1530x over previous
import functools

import jax
import jax.numpy as jnp
from jax import lax
from jax.experimental import pallas as pl
from jax.experimental.pallas import tpu as pltpu
from jax.experimental.pallas import tpu_sc as plsc

NNODE = 10000
NPAD = 10240
NWORKER = 32
CHUNK = 128
ROWS_PER_SUB = NPAD // 16


def _mm1_body(x_ref, w_ref, o_ref):
    o_ref[...] = jnp.dot(x_ref[...], w_ref[...],
                         preferred_element_type=jnp.float32)


def _mm1(x, W1):
    n, _ = x.shape
    d = W1.shape[1]
    return pl.pallas_call(
        _mm1_body,
        out_shape=jax.ShapeDtypeStruct((n, d), jnp.float32),
    )(x, W1)


def _mm2_body(p_ref, b1_ref, w2_ref, o_ref):
    h = jnp.maximum(p_ref[0] + p_ref[1] + b1_ref[...], 0.0)
    o_ref[...] = jnp.dot(h, w2_ref[...], preferred_element_type=jnp.float32)


def _mm2(p, b1, W2):
    n = p.shape[1]
    d = W2.shape[1]
    return pl.pallas_call(
        _mm2_body,
        out_shape=jax.ShapeDtypeStruct((n, d), jnp.float32),
    )(p, b1.reshape(1, -1), W2)


def _fin_body(q_ref, b2_ref, o_ref):
    o_ref[...] = q_ref[0] + q_ref[1] + b2_ref[...]


def _fin(q, b2):
    n, d = q.shape[1], q.shape[2]
    return pl.pallas_call(
        _fin_body,
        out_shape=jax.ShapeDtypeStruct((n, d), jnp.float32),
    )(q, b2.reshape(1, -1))


def _make_spmm(d, n_chunks):
    nv = d // 16
    mesh = plsc.VectorSubcoreMesh(core_axis_name="c", subcore_axis_name="s")

    @functools.partial(
        pl.kernel,
        out_type=jax.ShapeDtypeStruct((2 * NPAD, d), jnp.float32),
        mesh=mesh,
        compiler_params=pltpu.CompilerParams(use_tc_tiling_on_sc=False),
        scratch_types=[
            pltpu.VMEM((n_chunks, CHUNK), jnp.int32),
            pltpu.VMEM((n_chunks, CHUNK), jnp.int32),
            pltpu.VMEM((n_chunks * CHUNK,), jnp.float32),
            pltpu.VMEM((CHUNK, d), jnp.float32),
            pltpu.VMEM((CHUNK, d), jnp.float32),
            pltpu.VMEM((CHUNK, d), jnp.float32),
            pltpu.VMEM_SHARED((NPAD, d), jnp.float32),
            pltpu.SemaphoreType.DMA,
            pltpu.SemaphoreType.DMA,
            pltpu.SemaphoreType.DMA,
            pltpu.SemaphoreType.DMA,
        ],
    )
    def spmm(h_hbm, src_hbm, dst_hbm, w_hbm, out_hbm,
             src_v, dst_v, w_v, gbufa, gbufb, zbuf, acc,
             gsema, gsemb, ssema, ssemb):
        c = lax.axis_index("c")
        s = lax.axis_index("s")
        wid = s * 2 + c

        pltpu.sync_copy(src_hbm.at[wid], src_v)
        pltpu.sync_copy(dst_hbm.at[wid], dst_v)
        pltpu.sync_copy(w_hbm.at[wid], w_v)

        zero16 = jnp.zeros((16,), jnp.float32)

        def zc(i, carry):
            zbuf[i // nv, pl.ds((i % nv) * 16, 16)] = zero16
            return carry

        lax.fori_loop(0, CHUNK * nv, zc, 0)

        base_row = s * ROWS_PER_SUB
        for t in range(ROWS_PER_SUB // CHUNK):
            r0 = base_row + t * CHUNK
            pltpu.sync_copy(zbuf, acc.at[pl.ds(r0, CHUNK)])
        pltpu.async_copy(h_hbm.at[src_v.at[0]], gbufa, gsema)
        pltpu.async_copy(h_hbm.at[src_v.at[1]], gbufb, gsemb)
        plsc.subcore_barrier()

        def scale(ci, buf):
            def group_body(g, carry2):
                wchunk = w_v[pl.ds(ci * CHUNK + g * 16, 16)]
                for e16 in range(16):
                    wv = wchunk.at[jnp.full((16,), e16, jnp.int32)].get(
                        mode="promise_in_bounds")
                    row = g * 16 + e16
                    for j in range(nv):
                        sl = pl.ds(j * 16, 16)
                        buf[row, sl] = buf[row, sl] * wv
                return carry2

            lax.fori_loop(0, CHUNK // 16, group_body, 0)

        def pair_body(cj, carry):
            ci0 = 2 * cj
            ci1 = ci0 + 1
            for ci, buf, gsem, ssem in ((ci0, gbufa, gsema, ssema),
                                        (ci1, gbufb, gsemb, ssemb)):
                pltpu.make_async_copy(
                    h_hbm.at[src_v.at[ci]], buf, gsem).wait()
                scale(ci, buf)
                pltpu.async_copy(buf, acc.at[dst_v.at[ci]], ssem, add=True)
            for ci, buf, gsem, ssem in ((ci0, gbufa, gsema, ssema),
                                        (ci1, gbufb, gsemb, ssemb)):
                pltpu.make_async_copy(
                    buf, acc.at[dst_v.at[ci]], ssem).wait()

                @pl.when(ci + 2 < n_chunks)
                def _():
                    pltpu.async_copy(
                        h_hbm.at[src_v.at[ci + 2]], buf, gsem)

            return carry

        lax.fori_loop(0, n_chunks // 2, pair_body, 0)
        plsc.subcore_barrier()

        for t in range(ROWS_PER_SUB // CHUNK):
            r0 = base_row + t * CHUNK
            pltpu.sync_copy(acc.at[pl.ds(r0, CHUNK)], gbufa)
            pltpu.sync_copy(gbufa, out_hbm.at[pl.ds(c * NPAD + r0, CHUNK)])

    return spmm


def kernel(x, edge_index, edge_weight, W1, b1, W2, b2):
    src = edge_index[0].astype(jnp.int32)
    dst = edge_index[1].astype(jnp.int32)
    w = edge_weight.astype(jnp.float32)
    e = src.shape[0]

    n_chunks = -(-e // (NWORKER * CHUNK))
    n_chunks += n_chunks % 2
    e_pad = NWORKER * n_chunks * CHUNK
    pad = e_pad - e
    if pad:
        src = jnp.concatenate([src, jnp.zeros((pad,), jnp.int32)])
        dst = jnp.concatenate([dst, jnp.zeros((pad,), jnp.int32)])
        w = jnp.concatenate([w, jnp.zeros((pad,), jnp.float32)])
    src3 = src.reshape(NWORKER, n_chunks, CHUNK)
    dst3 = dst.reshape(NWORKER, n_chunks, CHUNK)
    w3 = w.reshape(NWORKER, n_chunks * CHUNK)

    nhid = W1.shape[1]
    ncls = W2.shape[1]

    xp = jnp.pad(x, ((0, NPAD - NNODE), (0, 0)))

    h = _mm1(xp, W1)
    p = _make_spmm(nhid, n_chunks)(h, src3, dst3, w3)
    h2 = _mm2(p.reshape(2, NPAD, nhid), b1, W2)
    q = _make_spmm(ncls, n_chunks)(h2, src3, dst3, w3)
    q = q.reshape(2, NPAD, ncls)[:, :NNODE, :]
    return _fin(q, b2)
